# Initial kernel scaffold; baseline (speedup 1.0000x reference)
import jax
import jax.numpy as jnp
from jax.experimental import pallas as pl


def kernel(x, Wg, W1, W2):
    raise NotImplementedError("write your pallas kernel here")


# speedup vs baseline: 1.2159x; 1.2159x over previous
import functools

import jax
import jax.numpy as jnp
from jax.experimental import pallas as pl
from jax.experimental.pallas import tpu as pltpu


def _router_body(x_ref, wg_ref, c_ref):
    x = x_ref[...]
    wg = wg_ref[...]
    logits = jnp.dot(x, wg, preferred_element_type=jnp.float32)
    m = jnp.max(logits, axis=-1, keepdims=True)
    ex = jnp.exp(logits - m)
    probs = ex / jnp.sum(ex, axis=-1, keepdims=True)
    e_iota = jax.lax.broadcasted_iota(jnp.int32, probs.shape, 1)
    n_e = probs.shape[-1]
    p1 = jnp.max(probs, axis=-1, keepdims=True)
    i1 = jnp.min(jnp.where(probs == p1, e_iota, n_e), axis=-1, keepdims=True)
    masked = jnp.where(e_iota == i1, -jnp.inf, probs)
    p2 = jnp.max(masked, axis=-1, keepdims=True)
    i2 = jnp.min(jnp.where(masked == p2, e_iota, n_e), axis=-1, keepdims=True)
    s = p1 + p2
    c = (jnp.where(e_iota == i1, p1 / s, 0.0)
         + jnp.where(e_iota == i2, p2 / s, 0.0))
    c_ref[...] = c.astype(jnp.float32)


def _router(x, wg, *, block_t):
    t, d = x.shape
    e = wg.shape[1]
    return pl.pallas_call(
        _router_body,
        grid=(t // block_t,),
        in_specs=[
            pl.BlockSpec((block_t, d), lambda i: (i, 0)),
            pl.BlockSpec((d, e), lambda i: (0, 0)),
        ],
        out_specs=pl.BlockSpec((block_t, e), lambda i: (i, 0)),
        out_shape=jax.ShapeDtypeStruct((t, e), jnp.float32),
    )(x, wg)


def _ffn_body(x_ref, w1_ref, w2_ref, c_ref, out_ref):
    e = pl.program_id(1)
    f = pl.program_id(2)
    x = x_ref[...]
    w1 = w1_ref[0]
    h = jnp.dot(x, w1, preferred_element_type=jnp.float32)
    h = jnp.maximum(h, 0.0).astype(jnp.bfloat16)
    w2 = w2_ref[0]
    y = jnp.dot(h, w2, preferred_element_type=jnp.float32)
    c = c_ref[...]
    e_iota = jax.lax.broadcasted_iota(jnp.int32, c.shape, 1)
    ccol = jnp.sum(jnp.where(e_iota == e, c, 0.0), axis=1, keepdims=True)
    contrib = y * ccol

    @pl.when((e == 0) & (f == 0))
    def _():
        out_ref[...] = contrib

    @pl.when((e > 0) | (f > 0))
    def _():
        out_ref[...] += contrib


def _ffn(xb, w1b, w2b, c, *, block_t, block_f):
    t, d = xb.shape
    n_e, _, f_dim = w1b.shape
    grid = (t // block_t, n_e, f_dim // block_f)
    return pl.pallas_call(
        _ffn_body,
        grid=grid,
        in_specs=[
            pl.BlockSpec((block_t, d), lambda i, e, f: (i, 0)),
            pl.BlockSpec((1, d, block_f), lambda i, e, f: (e, 0, f)),
            pl.BlockSpec((1, block_f, d), lambda i, e, f: (e, f, 0)),
            pl.BlockSpec((block_t, n_e), lambda i, e, f: (i, 0)),
        ],
        out_specs=pl.BlockSpec((block_t, d), lambda i, e, f: (i, 0)),
        out_shape=jax.ShapeDtypeStruct((t, d), jnp.float32),
        compiler_params=pltpu.CompilerParams(
            dimension_semantics=("parallel", "arbitrary", "arbitrary"),
        ),
    )(xb, w1b, w2b, c)


def kernel(x, Wg, W1, W2):
    t = x.shape[0]
    c = _router(x, Wg, block_t=min(1024, t))
    xb = x.astype(jnp.bfloat16)
    w1b = W1.astype(jnp.bfloat16)
    w2b = W2.astype(jnp.bfloat16)
    return _ffn(xb, w1b, w2b, c, block_t=min(1024, t), block_f=1024)
